# Initial kernel scaffold; baseline (speedup 1.0000x reference)
import jax
import jax.numpy as jnp
from jax.experimental import pallas as pl


def kernel(x, W, b):
    raise NotImplementedError("write your pallas kernel here")


# speedup vs baseline: 2.4702x; 2.4702x over previous
import functools

import jax
import jax.numpy as jnp
from jax.experimental import pallas as pl
from jax.experimental.pallas import tpu as pltpu

N_TOKENS = 16384
D_IN = 4096
N_EXPERTS = 64
BM = 512


def _gate_kernel(x_ref, wt_ref, b_ref, tp_ref, ent_ref, logits_ref):
    l = jnp.dot(x_ref[...], wt_ref[...], preferred_element_type=jnp.float32)
    l = l + b_ref[...]
    logits_ref[...] = l

    idx = jax.lax.broadcasted_iota(jnp.int32, l.shape, 1)
    neg_inf = jnp.float32(-jnp.inf)

    m1 = jnp.max(l, axis=1, keepdims=True)
    i1 = jnp.min(jnp.where(l == m1, idx, N_EXPERTS), axis=1, keepdims=True)
    l2 = jnp.where(idx == i1, neg_inf, l)
    m2 = jnp.max(l2, axis=1, keepdims=True)
    i2 = jnp.min(jnp.where(l2 == m2, idx, N_EXPERTS), axis=1, keepdims=True)

    e = jnp.exp(l - m1)
    s = jnp.sum(e, axis=1, keepdims=True)
    p = e / s

    mask = (idx == i1) | (idx == i2)
    tp = jnp.where(mask, p, 0.0)
    denom = jnp.sum(tp, axis=1, keepdims=True) + 1e-9
    tpn = tp / denom
    tp_ref[...] = tpn

    plogp = jnp.where(mask, tpn * jnp.log(jnp.maximum(tpn, 1e-12)), 0.0)
    ent_ref[...] = -jnp.sum(plogp, axis=1, keepdims=True)


@functools.partial(jax.jit, static_argnames=())
def kernel(x, W, b):
    wt = W.T
    b2 = b.reshape(1, N_EXPERTS)
    grid = (N_TOKENS // BM,)
    tp, ent, logits = pl.pallas_call(
        _gate_kernel,
        grid=grid,
        in_specs=[
            pl.BlockSpec((BM, D_IN), lambda i: (i, 0)),
            pl.BlockSpec((D_IN, N_EXPERTS), lambda i: (0, 0)),
            pl.BlockSpec((1, N_EXPERTS), lambda i: (0, 0)),
        ],
        out_specs=[
            pl.BlockSpec((BM, N_EXPERTS), lambda i: (i, 0)),
            pl.BlockSpec((BM, 1), lambda i: (i, 0)),
            pl.BlockSpec((BM, N_EXPERTS), lambda i: (i, 0)),
        ],
        out_shape=[
            jax.ShapeDtypeStruct((N_TOKENS, N_EXPERTS), jnp.float32),
            jax.ShapeDtypeStruct((N_TOKENS, 1), jnp.float32),
            jax.ShapeDtypeStruct((N_TOKENS, N_EXPERTS), jnp.float32),
        ],
        compiler_params=pltpu.CompilerParams(
            dimension_semantics=("arbitrary",),
        ),
    )(x, wt, b2)
    return tp, ent.reshape(N_TOKENS), logits


# speedup vs baseline: 2.6563x; 1.0754x over previous
import functools

import jax
import jax.numpy as jnp
from jax.experimental import pallas as pl
from jax.experimental.pallas import tpu as pltpu

N_TOKENS = 16384
D_IN = 4096
N_EXPERTS = 64
BM = 1024


def _gate_kernel(x_ref, wt_ref, b_ref, tp_ref, ent_ref, logits_ref):
    l = jnp.dot(x_ref[...], wt_ref[...], preferred_element_type=jnp.float32)
    l = l + b_ref[...]
    logits_ref[...] = l

    idx = jax.lax.broadcasted_iota(jnp.int32, l.shape, 1)
    neg_inf = jnp.float32(-jnp.inf)

    m1 = jnp.max(l, axis=1, keepdims=True)
    i1 = jnp.min(jnp.where(l == m1, idx, N_EXPERTS), axis=1, keepdims=True)
    l2 = jnp.where(idx == i1, neg_inf, l)
    m2 = jnp.max(l2, axis=1, keepdims=True)
    i2 = jnp.min(jnp.where(l2 == m2, idx, N_EXPERTS), axis=1, keepdims=True)

    e = jnp.exp(l - m1)
    s = jnp.sum(e, axis=1, keepdims=True)
    p = e / s

    mask = (idx == i1) | (idx == i2)
    tp = jnp.where(mask, p, 0.0)
    denom = jnp.sum(tp, axis=1, keepdims=True) + 1e-9
    tpn = tp / denom
    tp_ref[...] = tpn

    plogp = jnp.where(mask, tpn * jnp.log(jnp.maximum(tpn, 1e-12)), 0.0)
    ent_ref[...] = -jnp.sum(plogp, axis=1, keepdims=True)


@functools.partial(jax.jit, static_argnames=())
def kernel(x, W, b):
    wt = W.T
    b2 = b.reshape(1, N_EXPERTS)
    grid = (N_TOKENS // BM,)
    tp, ent, logits = pl.pallas_call(
        _gate_kernel,
        grid=grid,
        in_specs=[
            pl.BlockSpec((BM, D_IN), lambda i: (i, 0)),
            pl.BlockSpec((D_IN, N_EXPERTS), lambda i: (0, 0)),
            pl.BlockSpec((1, N_EXPERTS), lambda i: (0, 0)),
        ],
        out_specs=[
            pl.BlockSpec((BM, N_EXPERTS), lambda i: (i, 0)),
            pl.BlockSpec((BM, 1), lambda i: (i, 0)),
            pl.BlockSpec((BM, N_EXPERTS), lambda i: (i, 0)),
        ],
        out_shape=[
            jax.ShapeDtypeStruct((N_TOKENS, N_EXPERTS), jnp.float32),
            jax.ShapeDtypeStruct((N_TOKENS, 1), jnp.float32),
            jax.ShapeDtypeStruct((N_TOKENS, N_EXPERTS), jnp.float32),
        ],
        compiler_params=pltpu.CompilerParams(
            dimension_semantics=("arbitrary",),
        ),
    )(x, wt, b2)
    return tp, ent.reshape(N_TOKENS), logits


# speedup vs baseline: 2.6735x; 1.0065x over previous
import functools

import jax
import jax.numpy as jnp
from jax.experimental import pallas as pl
from jax.experimental.pallas import tpu as pltpu

N_TOKENS = 16384
D_IN = 4096
N_EXPERTS = 64
BM = 1024


def _gate_kernel(x_ref, wt_ref, b_ref, tp_ref, ent_ref, logits_ref):
    l = jnp.dot(x_ref[...], wt_ref[...], preferred_element_type=jnp.float32)
    l = l + b_ref[...]
    logits_ref[...] = l

    neg_inf = jnp.float32(-jnp.inf)

    m1 = jnp.max(l, axis=1, keepdims=True)
    is_max = l == m1
    c1 = jnp.sum(jnp.where(is_max, 1.0, 0.0), axis=1, keepdims=True)
    m2 = jnp.max(jnp.where(is_max, neg_inf, l), axis=1, keepdims=True)
    thresh = jnp.where(c1 >= 2.0, m1, m2)
    mask = l >= thresh

    e = jnp.exp(l - m1)
    te = jnp.where(mask, e, 0.0)
    denom = jnp.sum(te, axis=1, keepdims=True) + 1e-9
    tpn = te / denom
    tp_ref[...] = tpn

    plogp = tpn * jnp.log(jnp.maximum(tpn, 1e-12))
    ent_ref[...] = -jnp.sum(plogp, axis=1, keepdims=True)


@functools.partial(jax.jit, static_argnames=())
def kernel(x, W, b):
    wt = W.T
    b2 = b.reshape(1, N_EXPERTS)
    grid = (N_TOKENS // BM,)
    tp, ent, logits = pl.pallas_call(
        _gate_kernel,
        grid=grid,
        in_specs=[
            pl.BlockSpec((BM, D_IN), lambda i: (i, 0)),
            pl.BlockSpec((D_IN, N_EXPERTS), lambda i: (0, 0)),
            pl.BlockSpec((1, N_EXPERTS), lambda i: (0, 0)),
        ],
        out_specs=[
            pl.BlockSpec((BM, N_EXPERTS), lambda i: (i, 0)),
            pl.BlockSpec((BM, 1), lambda i: (i, 0)),
            pl.BlockSpec((BM, N_EXPERTS), lambda i: (i, 0)),
        ],
        out_shape=[
            jax.ShapeDtypeStruct((N_TOKENS, N_EXPERTS), jnp.float32),
            jax.ShapeDtypeStruct((N_TOKENS, 1), jnp.float32),
            jax.ShapeDtypeStruct((N_TOKENS, N_EXPERTS), jnp.float32),
        ],
        compiler_params=pltpu.CompilerParams(
            dimension_semantics=("arbitrary",),
        ),
    )(x, wt, b2)
    return tp, ent.reshape(N_TOKENS), logits
